# baseline (speedup 1.0000x reference)
import functools

import jax
import jax.numpy as jnp
from jax import lax
from jax.experimental import pallas as pl
from jax.experimental.pallas import tpu as pltpu
from jax.experimental.pallas import tpu_sc as plsc

_VOCAB_TILE = 2048


def _sc_gather(table, ids):
    n = ids.shape[0]
    d = table.shape[1]
    info = plsc.get_sparse_core_info()
    nc = info.num_cores
    half = n // nc
    mesh = plsc.ScalarSubcoreMesh(axis_name="c", num_cores=nc)

    @functools.partial(
        pl.kernel,
        mesh=mesh,
        out_type=jax.ShapeDtypeStruct((n, d), table.dtype),
        scratch_types=[
            pltpu.SMEM((half,), jnp.int32),
            pltpu.SemaphoreType.DMA,
            pltpu.SemaphoreType.DMA,
        ],
        compiler_params=pltpu.CompilerParams(use_tc_tiling_on_sc=True),
    )
    def gather_kernel(table_hbm, idx_hbm, out_hbm, idx_s, isem, sem):
        cid = lax.axis_index("c")
        base = cid * half
        pltpu.async_copy(idx_hbm.at[pl.ds(base, half)], idx_s, isem).wait()

        @pl.loop(0, half)
        def _(i):
            pltpu.async_copy(table_hbm.at[idx_s[i]], out_hbm.at[base + i], sem)

        @pl.loop(0, half)
        def _(i):
            pltpu.make_async_copy(
                table_hbm.at[idx_s[i]], out_hbm.at[base + i], sem
            ).wait()

    return gather_kernel(table, ids)


def _tc_project(x, W, b2):
    n, d = x.shape
    v = W.shape[1]

    def mm_kernel(x_ref, w_ref, b_ref, o_ref, xs_ref):
        @pl.when(pl.program_id(0) == 0)
        def _():
            xs_ref[...] = x_ref[...].astype(jnp.bfloat16)

        o_ref[...] = (
            jnp.dot(
                xs_ref[...],
                w_ref[...].astype(jnp.bfloat16),
                preferred_element_type=jnp.float32,
            )
            + b_ref[...]
        )

    return pl.pallas_call(
        mm_kernel,
        grid=(pl.cdiv(v, _VOCAB_TILE),),
        in_specs=[
            pl.BlockSpec((n, d), lambda i: (0, 0)),
            pl.BlockSpec((d, _VOCAB_TILE), lambda i: (0, i)),
            pl.BlockSpec((1, _VOCAB_TILE), lambda i: (0, i)),
        ],
        out_specs=pl.BlockSpec((n, _VOCAB_TILE), lambda i: (0, i)),
        out_shape=jax.ShapeDtypeStruct((n, v), jnp.float32),
        scratch_shapes=[pltpu.VMEM((n, d), jnp.bfloat16)],
    )(x, W, b2)


def kernel(input_ids, table, W, b):
    bsz, seq = input_ids.shape
    ids = input_ids.reshape(bsz * seq).astype(jnp.int32)
    x = _sc_gather(table, ids)
    logits = _tc_project(x, W, b.reshape(1, -1))
    return logits.reshape(bsz, seq, -1)


# speedup vs baseline: 1.0463x; 1.0463x over previous
import functools

import jax
import jax.numpy as jnp
from jax import lax
from jax.experimental import pallas as pl
from jax.experimental.pallas import tpu as pltpu
from jax.experimental.pallas import tpu_sc as plsc

_VOCAB_TILE = 2048


def _sc_gather_permuted(table, ids, seq):
    n = ids.shape[0]
    d = table.shape[1]
    bsz = n // seq
    info = plsc.get_sparse_core_info()
    nc = info.num_cores
    half = n // nc
    mesh = plsc.ScalarSubcoreMesh(axis_name="c", num_cores=nc)

    @functools.partial(
        pl.kernel,
        mesh=mesh,
        out_type=jax.ShapeDtypeStruct((n, d), table.dtype),
        scratch_types=[
            pltpu.SMEM((half,), jnp.int32),
            pltpu.SemaphoreType.DMA,
            pltpu.SemaphoreType.DMA,
        ],
    )
    def gather_kernel(table_hbm, idx_hbm, out_hbm, idx_s, isem, sem):
        cid = lax.axis_index("c")
        base = cid * half
        pltpu.async_copy(idx_hbm.at[pl.ds(base, half)], idx_s, isem).wait()

        @pl.loop(0, half)
        def _(i):
            src = base + i
            dst = (src % seq) * bsz + src // seq
            pltpu.async_copy(table_hbm.at[idx_s[i]], out_hbm.at[dst], sem)

        @pl.loop(0, half)
        def _(i):
            src = base + i
            dst = (src % seq) * bsz + src // seq
            pltpu.make_async_copy(
                table_hbm.at[idx_s[i]], out_hbm.at[dst], sem
            ).wait()

    return gather_kernel(table, ids)


def _tc_project_t(xp, W, b2, seq):
    n, d = xp.shape
    bsz = n // seq
    v = W.shape[1]

    def mm_kernel(x_ref, w_ref, b_ref, o_ref):
        xs = x_ref[...].astype(jnp.bfloat16)
        wt = w_ref[...].astype(jnp.bfloat16)
        acc = lax.dot_general(
            wt, xs,
            dimension_numbers=(((0,), (1,)), ((), ())),
            preferred_element_type=jnp.float32,
        )
        bias = lax.dot_general(
            b_ref[...], jnp.ones((1, bsz), jnp.float32),
            dimension_numbers=(((0,), (0,)), ((), ())),
            preferred_element_type=jnp.float32,
        )
        o_ref[0] = acc + bias

    return pl.pallas_call(
        mm_kernel,
        grid=(pl.cdiv(v, _VOCAB_TILE), seq),
        in_specs=[
            pl.BlockSpec((bsz, d), lambda i, l: (l, 0)),
            pl.BlockSpec((d, _VOCAB_TILE), lambda i, l: (0, i)),
            pl.BlockSpec((1, _VOCAB_TILE), lambda i, l: (0, i)),
        ],
        out_specs=pl.BlockSpec((1, _VOCAB_TILE, bsz), lambda i, l: (l, i, 0)),
        out_shape=jax.ShapeDtypeStruct((seq, v, bsz), jnp.float32),
    )(xp, W, b2)


def kernel(input_ids, table, W, b):
    bsz, seq = input_ids.shape
    ids = input_ids.reshape(bsz * seq).astype(jnp.int32)
    xp = _sc_gather_permuted(table, ids, seq)
    logits_t = _tc_project_t(xp, W, b.reshape(1, -1), seq)
    return logits_t.transpose(2, 0, 1)
